# Initial kernel scaffold; baseline (speedup 1.0000x reference)
import jax
import jax.numpy as jnp
from jax.experimental import pallas as pl


def kernel(x, edge_index, confidences, params):
    raise NotImplementedError("write your pallas kernel here")


# speedup vs baseline: 6.9394x; 6.9394x over previous
import functools

import jax
import jax.numpy as jnp
from jax import lax
from jax.experimental import pallas as pl
from jax.experimental.pallas import tpu as pltpu
from jax.experimental.pallas import tpu_sc as plsc

N = 10000
E = 320000
D = 128

NC = 2
NS = 16
NW = NC * NS
EPT = E // NW
RPT = 624
REM = N - NS * RPT
K = 80
NCHUNK = EPT // K
ZR = 208

_mesh = plsc.VectorSubcoreMesh(core_axis_name="c", subcore_axis_name="s")
_sc_params = pltpu.CompilerParams(needs_layout_passes=False)


def _edge_weights(conf, src, dst):
    @functools.partial(
        pl.kernel,
        out_type=jax.ShapeDtypeStruct((E,), jnp.float32),
        mesh=_mesh,
        compiler_params=_sc_params,
        scratch_types=[
            pltpu.VMEM((N,), jnp.float32),
            pltpu.VMEM((EPT,), jnp.int32),
            pltpu.VMEM((EPT,), jnp.int32),
            pltpu.VMEM((EPT,), jnp.float32),
        ],
    )
    def kern(conf_hbm, src_hbm, dst_hbm, out_hbm, conf_v, src_v, dst_v, w_v):
        cid = lax.axis_index("c")
        sid = lax.axis_index("s")
        base = (sid * NC + cid) * EPT
        pltpu.sync_copy(conf_hbm, conf_v)
        pltpu.sync_copy(src_hbm.at[pl.ds(base, EPT)], src_v)
        pltpu.sync_copy(dst_hbm.at[pl.ds(base, EPT)], dst_v)

        def body(i, carry):
            sl = pl.ds(i * 16, 16)
            cs = plsc.load_gather(conf_v, [src_v[sl]])
            cd = plsc.load_gather(conf_v, [dst_v[sl]])
            w_v[sl] = jnp.exp(-jnp.abs(cs - cd))
            return carry

        lax.fori_loop(0, EPT // 16, body, 0)
        pltpu.sync_copy(w_v, out_hbm.at[pl.ds(base, EPT)])

    return kern(conf, src, dst)


def _aggregate(msg, src, dst, w):
    @functools.partial(
        pl.kernel,
        out_type=jax.ShapeDtypeStruct((NC, N, D), jnp.float32),
        mesh=_mesh,
        compiler_params=_sc_params,
        scratch_types=[
            pltpu.VMEM_SHARED((N, D), jnp.float32),
            pltpu.VMEM((K,), jnp.int32),
            pltpu.VMEM((K,), jnp.int32),
            pltpu.VMEM((K,), jnp.float32),
            pltpu.VMEM((K, D), jnp.float32),
            pltpu.VMEM((ZR, D), jnp.float32),
            pltpu.SemaphoreType.DMA,
        ],
    )
    def kern(msg_hbm, src_hbm, dst_hbm, w_hbm, out_hbm,
             acc, sidx, didx, wv, rows, zbuf, sem):
        cid = lax.axis_index("c")
        sid = lax.axis_index("s")
        wid = sid * NC + cid

        def zrow(i, carry):
            for c in range(D // 16):
                zbuf[i, pl.ds(c * 16, 16)] = jnp.zeros((16,), jnp.float32)
            return carry

        lax.fori_loop(0, ZR, zrow, 0)
        for r in range(RPT // ZR):
            pltpu.sync_copy(zbuf, acc.at[pl.ds(sid * RPT + r * ZR, ZR)])

        @pl.when(sid == NS - 1)
        def _():
            pltpu.sync_copy(zbuf.at[pl.ds(0, REM)],
                            acc.at[pl.ds(NS * RPT, REM)])

        plsc.subcore_barrier()

        ebase = wid * EPT

        def chunk(j, carry):
            base = ebase + j * K
            pltpu.sync_copy(src_hbm.at[pl.ds(base, K)], sidx)
            pltpu.sync_copy(dst_hbm.at[pl.ds(base, K)], didx)
            pltpu.sync_copy(w_hbm.at[pl.ds(base, K)], wv)
            pltpu.async_copy(msg_hbm.at[sidx], rows, sem).wait()

            def scale(g, c2):
                wvec = wv[pl.ds(g * 16, 16)]
                for r in range(16):
                    wi = wvec[r]
                    for c in range(D // 16):
                        sl = pl.ds(c * 16, 16)
                        rows[g * 16 + r, sl] = rows[g * 16 + r, sl] * wi
                return c2

            lax.fori_loop(0, K // 16, scale, 0)
            pltpu.sync_copy(rows, acc.at[didx], add=True)
            return carry

        lax.fori_loop(0, NCHUNK, chunk, 0)
        plsc.subcore_barrier()
        rbase = sid * RPT
        pltpu.sync_copy(acc.at[pl.ds(rbase, RPT)],
                        out_hbm.at[cid, pl.ds(rbase, RPT)])

        @pl.when(sid == NS - 1)
        def _():
            pltpu.sync_copy(acc.at[pl.ds(NS * RPT, REM)],
                            out_hbm.at[cid, pl.ds(NS * RPT, REM)])

    return kern(msg, src, dst, w)


_TC_R = 1000


def _dense_layer(h, mw1, mb1, mw2, mb2, sw1, sb1, sw2, sb2):
    def body(h_ref, mw1r, mb1r, mw2r, mb2r, sw1r, sb1r, sw2r, sb2r,
             msg_ref, sl_ref):
        hb = h_ref[...]
        hid = jnp.maximum(
            jnp.dot(hb, mw1r[...], preferred_element_type=jnp.float32)
            + mb1r[...], 0.0)
        msg_ref[...] = (
            jnp.dot(hid, mw2r[...], preferred_element_type=jnp.float32)
            + mb2r[...])
        shid = jnp.maximum(
            jnp.dot(hb, sw1r[...], preferred_element_type=jnp.float32)
            + sb1r[...], 0.0)
        sl_ref[...] = (
            jnp.dot(shid, sw2r[...], preferred_element_type=jnp.float32)
            + sb2r[...])

    row_spec = pl.BlockSpec((_TC_R, D), lambda i: (i, 0))
    w_spec = pl.BlockSpec((D, D), lambda i: (0, 0))
    b_spec = pl.BlockSpec((1, D), lambda i: (0, 0))
    return pl.pallas_call(
        body,
        grid=(N // _TC_R,),
        in_specs=[row_spec, w_spec, b_spec, w_spec, b_spec,
                  w_spec, b_spec, w_spec, b_spec],
        out_specs=[row_spec, row_spec],
        out_shape=[jax.ShapeDtypeStruct((N, D), jnp.float32),
                   jax.ShapeDtypeStruct((N, D), jnp.float32)],
    )(h, mw1, mb1.reshape(1, D), mw2, mb2.reshape(1, D),
      sw1, sb1.reshape(1, D), sw2, sb2.reshape(1, D))


def _combine(a0, a1, slp):
    def body(a0r, a1r, slr, outr):
        outr[...] = jnp.maximum(a0r[...] + a1r[...] + slr[...], 0.0)

    row_spec = pl.BlockSpec((_TC_R, D), lambda i: (i, 0))
    return pl.pallas_call(
        body,
        grid=(N // _TC_R,),
        in_specs=[row_spec, row_spec, row_spec],
        out_specs=row_spec,
        out_shape=jax.ShapeDtypeStruct((N, D), jnp.float32),
    )(a0, a1, slp)


def _importance(h1, h2, h3, iw1, ib1, iw2t):
    def body(h1r, h2r, h3r, w1r, b1r, w2r, outr):
        w1, b1, w2 = w1r[...], b1r[...], w2r[...]

        def score(hb):
            sh = jnp.maximum(
                jnp.dot(hb, w1, preferred_element_type=jnp.float32) + b1, 0.0)
            return jnp.sum(sh * w2, axis=1, keepdims=True)

        a, b, c = h1r[...], h2r[...], h3r[...]
        s1, s2, s3 = score(a), score(b), score(c)
        m = jnp.maximum(jnp.maximum(s1, s2), s3)
        e1 = jnp.exp(s1 - m)
        e2 = jnp.exp(s2 - m)
        e3 = jnp.exp(s3 - m)
        outr[...] = (e1 * a + e2 * b + e3 * c) / (e1 + e2 + e3)

    row_spec = pl.BlockSpec((_TC_R, D), lambda i: (i, 0))
    w_spec = pl.BlockSpec((D, D), lambda i: (0, 0))
    b_spec = pl.BlockSpec((1, D), lambda i: (0, 0))
    return pl.pallas_call(
        body,
        grid=(N // _TC_R,),
        in_specs=[row_spec, row_spec, row_spec, w_spec, b_spec, b_spec],
        out_specs=row_spec,
        out_shape=jax.ShapeDtypeStruct((N, D), jnp.float32),
    )(h1, h2, h3, iw1, ib1, iw2t)


def kernel(x, edge_index, confidences, params):
    src = edge_index[0]
    dst = edge_index[1]
    w = _edge_weights(confidences, src, dst)
    h = x
    outs = []
    for lp in params["layers"]:
        msg, slp = _dense_layer(h, lp["mw1"], lp["mb1"], lp["mw2"], lp["mb2"],
                                lp["sw1"], lp["sb1"], lp["sw2"], lp["sb2"])
        parts = _aggregate(msg, src, dst, w)
        h = _combine(parts[0], parts[1], slp)
        outs.append(h)
    return _importance(outs[0], outs[1], outs[2],
                       params["imp_w1"], params["imp_b1"].reshape(1, D),
                       params["imp_w2"].reshape(1, D))


# speedup vs baseline: 16.8185x; 2.4236x over previous
import functools

import jax
import jax.numpy as jnp
from jax import lax
from jax.experimental import pallas as pl
from jax.experimental.pallas import tpu as pltpu
from jax.experimental.pallas import tpu_sc as plsc

N = 10000
E = 320000
D = 128

NC = 2
NS = 16
NW = NC * NS
EPT = E // NW
RPT = 624
REM = N - NS * RPT
K = 80
NCHUNK = EPT // K

_mesh = plsc.VectorSubcoreMesh(core_axis_name="c", subcore_axis_name="s")
_sc_params = pltpu.CompilerParams(needs_layout_passes=False)


def _edge_weights(conf, src, dst):
    @functools.partial(
        pl.kernel,
        out_type=jax.ShapeDtypeStruct((E,), jnp.float32),
        mesh=_mesh,
        compiler_params=_sc_params,
        scratch_types=[
            pltpu.VMEM((N,), jnp.float32),
            pltpu.VMEM((EPT,), jnp.int32),
            pltpu.VMEM((EPT,), jnp.int32),
            pltpu.VMEM((EPT,), jnp.float32),
        ],
    )
    def kern(conf_hbm, src_hbm, dst_hbm, out_hbm, conf_v, src_v, dst_v, w_v):
        cid = lax.axis_index("c")
        sid = lax.axis_index("s")
        base = (sid * NC + cid) * EPT
        pltpu.sync_copy(conf_hbm, conf_v)
        pltpu.sync_copy(src_hbm.at[pl.ds(base, EPT)], src_v)
        pltpu.sync_copy(dst_hbm.at[pl.ds(base, EPT)], dst_v)

        def body(i, carry):
            sl = pl.ds(i * 16, 16)
            cs = plsc.load_gather(conf_v, [src_v[sl]])
            cd = plsc.load_gather(conf_v, [dst_v[sl]])
            w_v[sl] = jnp.exp(-jnp.abs(cs - cd))
            return carry

        lax.fori_loop(0, EPT // 16, body, 0)
        pltpu.sync_copy(w_v, out_hbm.at[pl.ds(base, EPT)])

    return kern(conf, src, dst)


def _aggregate(msg, src, dst3, w):
    @functools.partial(
        pl.kernel,
        out_type=jax.ShapeDtypeStruct((NC, N, D), jnp.float32),
        mesh=_mesh,
        compiler_params=_sc_params,
        scratch_types=[
            pltpu.VMEM_SHARED((N, D), jnp.float32),
            pltpu.VMEM((EPT,), jnp.int32),
            pltpu.VMEM((EPT,), jnp.int32),
            pltpu.VMEM((K,), jnp.float32),
            pltpu.VMEM((K,), jnp.float32),
            pltpu.VMEM((K, D), jnp.float32),
            pltpu.VMEM((K, D), jnp.float32),
            pltpu.SemaphoreType.DMA,
            pltpu.SemaphoreType.DMA,
            pltpu.SemaphoreType.DMA,
            pltpu.SemaphoreType.DMA,
        ],
    )
    def kern(msg_hbm, src_hbm, dst_hbm, w_hbm, out_hbm,
             acc, srcv, dstv, w_a, w_b, rows_a, rows_b,
             sem_a, sem_b, wsem_a, wsem_b):
        cid = lax.axis_index("c")
        sid = lax.axis_index("s")
        wid = sid * NC + cid
        ebase = wid * EPT
        pltpu.sync_copy(src_hbm.at[pl.ds(ebase, EPT)], srcv)
        pltpu.sync_copy(dst_hbm.at[pl.ds(ebase, EPT)], dstv)

        def zrow(i, carry):
            for c in range(D // 16):
                rows_a[i, pl.ds(c * 16, 16)] = jnp.zeros((16,), jnp.float32)
            return carry

        lax.fori_loop(0, K, zrow, 0)
        for r in range(RPT // K):
            pltpu.sync_copy(rows_a, acc.at[pl.ds(sid * RPT + r * K, K)])
        pltpu.sync_copy(rows_a.at[pl.ds(0, RPT - (RPT // K) * K)],
                        acc.at[pl.ds(sid * RPT + (RPT // K) * K,
                                     RPT - (RPT // K) * K)])

        @pl.when(sid == NS - 1)
        def _():
            pltpu.sync_copy(rows_a.at[pl.ds(0, REM)],
                            acc.at[pl.ds(NS * RPT, REM)])

        plsc.subcore_barrier()

        def start(j, buf, wbuf, sem, wsem):
            pltpu.async_copy(msg_hbm.at[srcv.at[pl.ds(j * K, K)]], buf, sem)
            pltpu.async_copy(w_hbm.at[pl.ds(ebase + j * K, K)], wbuf, wsem)

        def finish(j, buf, wbuf, sem, wsem):
            pltpu.make_async_copy(
                msg_hbm.at[srcv.at[pl.ds(j * K, K)]], buf, sem).wait()
            pltpu.make_async_copy(
                w_hbm.at[pl.ds(ebase + j * K, K)], wbuf, wsem).wait()

        def process(j, buf, wbuf):
            def scale(g, c2):
                wvec = wbuf[pl.ds(g * 16, 16)]
                for r in range(16):
                    wi = wvec[r]
                    for c in range(D // 16):
                        sl = pl.ds(c * 16, 16)
                        buf[g * 16 + r, sl] = buf[g * 16 + r, sl] * wi
                return c2

            lax.fori_loop(0, K // 16, scale, 0)
            pltpu.sync_copy(buf, acc.at[dstv.at[pl.ds(j * K, K)]], add=True)

        start(0, rows_a, w_a, sem_a, wsem_a)

        def pair(t, carry):
            j0 = 2 * t
            start(j0 + 1, rows_b, w_b, sem_b, wsem_b)
            finish(j0, rows_a, w_a, sem_a, wsem_a)
            process(j0, rows_a, w_a)
            start(j0 + 2, rows_a, w_a, sem_a, wsem_a)
            finish(j0 + 1, rows_b, w_b, sem_b, wsem_b)
            process(j0 + 1, rows_b, w_b)
            return carry

        lax.fori_loop(0, (NCHUNK - 1) // 2, pair, 0)
        finish(NCHUNK - 1, rows_a, w_a, sem_a, wsem_a)
        process(NCHUNK - 1, rows_a, w_a)

        plsc.subcore_barrier()
        rbase = sid * RPT
        pltpu.sync_copy(acc.at[pl.ds(rbase, RPT)],
                        out_hbm.at[cid, pl.ds(rbase, RPT)])

        @pl.when(sid == NS - 1)
        def _():
            pltpu.sync_copy(acc.at[pl.ds(NS * RPT, REM)],
                            out_hbm.at[cid, pl.ds(NS * RPT, REM)])

    return kern(msg, src, dst3, w)


_TC_R = 1000


def _dense_layer(h, mw1, mb1, mw2, mb2, sw1, sb1, sw2, sb2):
    def body(h_ref, mw1r, mb1r, mw2r, mb2r, sw1r, sb1r, sw2r, sb2r,
             msg_ref, sl_ref):
        hb = h_ref[...]
        hid = jnp.maximum(
            jnp.dot(hb, mw1r[...], preferred_element_type=jnp.float32)
            + mb1r[...], 0.0)
        msg_ref[...] = (
            jnp.dot(hid, mw2r[...], preferred_element_type=jnp.float32)
            + mb2r[...])
        shid = jnp.maximum(
            jnp.dot(hb, sw1r[...], preferred_element_type=jnp.float32)
            + sb1r[...], 0.0)
        sl_ref[...] = (
            jnp.dot(shid, sw2r[...], preferred_element_type=jnp.float32)
            + sb2r[...])

    row_spec = pl.BlockSpec((_TC_R, D), lambda i: (i, 0))
    w_spec = pl.BlockSpec((D, D), lambda i: (0, 0))
    b_spec = pl.BlockSpec((1, D), lambda i: (0, 0))
    return pl.pallas_call(
        body,
        grid=(N // _TC_R,),
        in_specs=[row_spec, w_spec, b_spec, w_spec, b_spec,
                  w_spec, b_spec, w_spec, b_spec],
        out_specs=[row_spec, row_spec],
        out_shape=[jax.ShapeDtypeStruct((N, D), jnp.float32),
                   jax.ShapeDtypeStruct((N, D), jnp.float32)],
    )(h, mw1, mb1.reshape(1, D), mw2, mb2.reshape(1, D),
      sw1, sb1.reshape(1, D), sw2, sb2.reshape(1, D))


def _combine(a0, a1, slp):
    def body(a0r, a1r, slr, outr):
        outr[...] = jnp.maximum(a0r[...] + a1r[...] + slr[...], 0.0)

    row_spec = pl.BlockSpec((_TC_R, D), lambda i: (i, 0))
    return pl.pallas_call(
        body,
        grid=(N // _TC_R,),
        in_specs=[row_spec, row_spec, row_spec],
        out_specs=row_spec,
        out_shape=jax.ShapeDtypeStruct((N, D), jnp.float32),
    )(a0, a1, slp)


def _importance(h1, h2, h3, iw1, ib1, iw2t):
    def body(h1r, h2r, h3r, w1r, b1r, w2r, outr):
        w1, b1, w2 = w1r[...], b1r[...], w2r[...]

        def score(hb):
            sh = jnp.maximum(
                jnp.dot(hb, w1, preferred_element_type=jnp.float32) + b1, 0.0)
            return jnp.sum(sh * w2, axis=1, keepdims=True)

        a, b, c = h1r[...], h2r[...], h3r[...]
        s1, s2, s3 = score(a), score(b), score(c)
        m = jnp.maximum(jnp.maximum(s1, s2), s3)
        e1 = jnp.exp(s1 - m)
        e2 = jnp.exp(s2 - m)
        e3 = jnp.exp(s3 - m)
        outr[...] = (e1 * a + e2 * b + e3 * c) / (e1 + e2 + e3)

    row_spec = pl.BlockSpec((_TC_R, D), lambda i: (i, 0))
    w_spec = pl.BlockSpec((D, D), lambda i: (0, 0))
    b_spec = pl.BlockSpec((1, D), lambda i: (0, 0))
    return pl.pallas_call(
        body,
        grid=(N // _TC_R,),
        in_specs=[row_spec, row_spec, row_spec, w_spec, b_spec, b_spec],
        out_specs=row_spec,
        out_shape=jax.ShapeDtypeStruct((N, D), jnp.float32),
    )(h1, h2, h3, iw1, ib1, iw2t)


def kernel(x, edge_index, confidences, params):
    src = edge_index[0]
    dst = edge_index[1]
    w = _edge_weights(confidences, src, dst)
    h = x
    outs = []
    for lp in params["layers"]:
        msg, slp = _dense_layer(h, lp["mw1"], lp["mb1"], lp["mw2"], lp["mb2"],
                                lp["sw1"], lp["sb1"], lp["sw2"], lp["sb2"])
        parts = _aggregate(msg, src, dst, w)
        h = _combine(parts[0], parts[1], slp)
        outs.append(h)
    return _importance(outs[0], outs[1], outs[2],
                       params["imp_w1"], params["imp_b1"].reshape(1, D),
                       params["imp_w2"].reshape(1, D))


# speedup vs baseline: 18.7615x; 1.1155x over previous
import functools

import jax
import jax.numpy as jnp
from jax import lax
from jax.experimental import pallas as pl
from jax.experimental.pallas import tpu as pltpu
from jax.experimental.pallas import tpu_sc as plsc

N = 10000
E = 320000
D = 128

NC = 2
NS = 16
NW = NC * NS
EPT = E // NW
RPT = 624
REM = N - NS * RPT
K = 80
NCHUNK = EPT // K

_mesh = plsc.VectorSubcoreMesh(core_axis_name="c", subcore_axis_name="s")
_sc_params = pltpu.CompilerParams(needs_layout_passes=False)


def _edge_weights(conf, src, dst):
    @functools.partial(
        pl.kernel,
        out_type=jax.ShapeDtypeStruct((E,), jnp.float32),
        mesh=_mesh,
        compiler_params=_sc_params,
        scratch_types=[
            pltpu.VMEM((N,), jnp.float32),
            pltpu.VMEM((EPT,), jnp.int32),
            pltpu.VMEM((EPT,), jnp.int32),
            pltpu.VMEM((EPT,), jnp.float32),
        ],
    )
    def kern(conf_hbm, src_hbm, dst_hbm, out_hbm, conf_v, src_v, dst_v, w_v):
        cid = lax.axis_index("c")
        sid = lax.axis_index("s")
        base = (sid * NC + cid) * EPT
        pltpu.sync_copy(conf_hbm, conf_v)
        pltpu.sync_copy(src_hbm.at[pl.ds(base, EPT)], src_v)
        pltpu.sync_copy(dst_hbm.at[pl.ds(base, EPT)], dst_v)

        def body(i, carry):
            sl = pl.ds(i * 16, 16)
            cs = plsc.load_gather(conf_v, [src_v[sl]])
            cd = plsc.load_gather(conf_v, [dst_v[sl]])
            w_v[sl] = jnp.exp(-jnp.abs(cs - cd))
            return carry

        lax.fori_loop(0, EPT // 16, body, 0)
        pltpu.sync_copy(w_v, out_hbm.at[pl.ds(base, EPT)])

    return kern(conf, src, dst)


def _aggregate(msg, src, dst3, w):
    @functools.partial(
        pl.kernel,
        out_type=jax.ShapeDtypeStruct((NC, N, D), jnp.float32),
        mesh=_mesh,
        compiler_params=_sc_params,
        scratch_types=[
            pltpu.VMEM_SHARED((N, D), jnp.float32),
            pltpu.VMEM((EPT,), jnp.int32),
            pltpu.VMEM((K, D), jnp.float32),
            pltpu.VMEM((K, D), jnp.float32),
            pltpu.VMEM((K, D), jnp.float32),
            pltpu.VMEM((K,), jnp.float32),
            pltpu.VMEM((K,), jnp.float32),
            pltpu.VMEM((K,), jnp.float32),
            pltpu.VMEM((K,), jnp.int32),
            pltpu.VMEM((K,), jnp.int32),
            pltpu.VMEM((K,), jnp.int32),
            pltpu.SemaphoreType.DMA,
            pltpu.SemaphoreType.DMA,
            pltpu.SemaphoreType.DMA,
            pltpu.SemaphoreType.DMA,
            pltpu.SemaphoreType.DMA,
            pltpu.SemaphoreType.DMA,
            pltpu.SemaphoreType.DMA,
            pltpu.SemaphoreType.DMA,
            pltpu.SemaphoreType.DMA,
        ],
    )
    def kern(msg_hbm, src_hbm, dst_hbm, w_hbm, out_hbm,
             acc, srcv, rows0, rows1, rows2, wb0, wb1, wb2, db0, db1, db2,
             gs0, gs1, gs2, ws0, ws1, ws2, ss0, ss1, ss2):
        cid = lax.axis_index("c")
        sid = lax.axis_index("s")
        wid = sid * NC + cid
        ebase = wid * EPT
        rows = (rows0, rows1, rows2)
        wb = (wb0, wb1, wb2)
        db = (db0, db1, db2)
        gs = (gs0, gs1, gs2)
        ws = (ws0, ws1, ws2)
        ss = (ss0, ss1, ss2)
        pltpu.sync_copy(src_hbm.at[pl.ds(ebase, EPT)], srcv)

        def zrow(i, carry):
            for c in range(D // 16):
                rows0[i, pl.ds(c * 16, 16)] = jnp.zeros((16,), jnp.float32)
            return carry

        lax.fori_loop(0, K, zrow, 0)
        for r in range(RPT // K):
            pltpu.sync_copy(rows0, acc.at[pl.ds(sid * RPT + r * K, K)])
        pltpu.sync_copy(rows0.at[pl.ds(0, RPT - (RPT // K) * K)],
                        acc.at[pl.ds(sid * RPT + (RPT // K) * K,
                                     RPT - (RPT // K) * K)])

        @pl.when(sid == NS - 1)
        def _():
            pltpu.sync_copy(rows0.at[pl.ds(0, REM)],
                            acc.at[pl.ds(NS * RPT, REM)])

        plsc.subcore_barrier()

        def start_fetch(j, s):
            pltpu.async_copy(msg_hbm.at[srcv.at[pl.ds(j * K, K)]],
                             rows[s], gs[s])
            pltpu.async_copy(w_hbm.at[pl.ds(ebase + j * K, K)], wb[s], ws[s])
            pltpu.async_copy(dst_hbm.at[pl.ds(ebase + j * K, K)], db[s], ws[s])

        def wait_fetch(j, s):
            pltpu.make_async_copy(msg_hbm.at[srcv.at[pl.ds(j * K, K)]],
                                  rows[s], gs[s]).wait()
            pltpu.make_async_copy(w_hbm.at[pl.ds(ebase + j * K, K)],
                                  wb[s], ws[s]).wait()
            pltpu.make_async_copy(dst_hbm.at[pl.ds(ebase + j * K, K)],
                                  db[s], ws[s]).wait()

        def start_scatter(s):
            pltpu.async_copy(rows[s], acc.at[db[s]], ss[s], add=True)

        def wait_scatter(s):
            pltpu.make_async_copy(rows[s], acc.at[db[s]], ss[s]).wait()

        def scale(s):
            def body(g, c2):
                wvec = wb[s][pl.ds(g * 16, 16)]
                for r in range(16):
                    wi = wvec[r]
                    for c in range(D // 16):
                        sl = pl.ds(c * 16, 16)
                        rows[s][g * 16 + r, sl] = rows[s][g * 16 + r, sl] * wi
                return c2

            lax.fori_loop(0, K // 16, body, 0)

        def step(j, s, first):
            wait_fetch(j, s)
            scale(s)
            start_scatter(s)
            if not first:
                wait_scatter((s + 2) % 3)

            @pl.when(j + 2 < NCHUNK)
            def _():
                start_fetch(j + 2, (s + 2) % 3)

        start_fetch(0, 0)
        start_fetch(1, 1)
        step(0, 0, True)
        step(1, 1, False)

        def triple(t, carry):
            j0 = 3 * t + 2
            step(j0, 2, False)
            step(j0 + 1, 0, False)
            step(j0 + 2, 1, False)
            return carry

        lax.fori_loop(0, (NCHUNK - 2) // 3, triple, 0)
        wait_scatter((NCHUNK - 1) % 3)

        plsc.subcore_barrier()
        rbase = sid * RPT
        pltpu.sync_copy(acc.at[pl.ds(rbase, RPT)],
                        out_hbm.at[cid, pl.ds(rbase, RPT)])

        @pl.when(sid == NS - 1)
        def _():
            pltpu.sync_copy(acc.at[pl.ds(NS * RPT, REM)],
                            out_hbm.at[cid, pl.ds(NS * RPT, REM)])

    return kern(msg, src, dst3, w)


_TC_R = 1000


def _dense_layer(h, mw1, mb1, mw2, mb2, sw1, sb1, sw2, sb2):
    def body(h_ref, mw1r, mb1r, mw2r, mb2r, sw1r, sb1r, sw2r, sb2r,
             msg_ref, sl_ref):
        hb = h_ref[...]
        hid = jnp.maximum(
            jnp.dot(hb, mw1r[...], preferred_element_type=jnp.float32)
            + mb1r[...], 0.0)
        msg_ref[...] = (
            jnp.dot(hid, mw2r[...], preferred_element_type=jnp.float32)
            + mb2r[...])
        shid = jnp.maximum(
            jnp.dot(hb, sw1r[...], preferred_element_type=jnp.float32)
            + sb1r[...], 0.0)
        sl_ref[...] = (
            jnp.dot(shid, sw2r[...], preferred_element_type=jnp.float32)
            + sb2r[...])

    row_spec = pl.BlockSpec((_TC_R, D), lambda i: (i, 0))
    w_spec = pl.BlockSpec((D, D), lambda i: (0, 0))
    b_spec = pl.BlockSpec((1, D), lambda i: (0, 0))
    return pl.pallas_call(
        body,
        grid=(N // _TC_R,),
        in_specs=[row_spec, w_spec, b_spec, w_spec, b_spec,
                  w_spec, b_spec, w_spec, b_spec],
        out_specs=[row_spec, row_spec],
        out_shape=[jax.ShapeDtypeStruct((N, D), jnp.float32),
                   jax.ShapeDtypeStruct((N, D), jnp.float32)],
    )(h, mw1, mb1.reshape(1, D), mw2, mb2.reshape(1, D),
      sw1, sb1.reshape(1, D), sw2, sb2.reshape(1, D))


def _combine(a0, a1, slp):
    def body(a0r, a1r, slr, outr):
        outr[...] = jnp.maximum(a0r[...] + a1r[...] + slr[...], 0.0)

    row_spec = pl.BlockSpec((_TC_R, D), lambda i: (i, 0))
    return pl.pallas_call(
        body,
        grid=(N // _TC_R,),
        in_specs=[row_spec, row_spec, row_spec],
        out_specs=row_spec,
        out_shape=jax.ShapeDtypeStruct((N, D), jnp.float32),
    )(a0, a1, slp)


def _importance(h1, h2, h3, iw1, ib1, iw2t):
    def body(h1r, h2r, h3r, w1r, b1r, w2r, outr):
        w1, b1, w2 = w1r[...], b1r[...], w2r[...]

        def score(hb):
            sh = jnp.maximum(
                jnp.dot(hb, w1, preferred_element_type=jnp.float32) + b1, 0.0)
            return jnp.sum(sh * w2, axis=1, keepdims=True)

        a, b, c = h1r[...], h2r[...], h3r[...]
        s1, s2, s3 = score(a), score(b), score(c)
        m = jnp.maximum(jnp.maximum(s1, s2), s3)
        e1 = jnp.exp(s1 - m)
        e2 = jnp.exp(s2 - m)
        e3 = jnp.exp(s3 - m)
        outr[...] = (e1 * a + e2 * b + e3 * c) / (e1 + e2 + e3)

    row_spec = pl.BlockSpec((_TC_R, D), lambda i: (i, 0))
    w_spec = pl.BlockSpec((D, D), lambda i: (0, 0))
    b_spec = pl.BlockSpec((1, D), lambda i: (0, 0))
    return pl.pallas_call(
        body,
        grid=(N // _TC_R,),
        in_specs=[row_spec, row_spec, row_spec, w_spec, b_spec, b_spec],
        out_specs=row_spec,
        out_shape=jax.ShapeDtypeStruct((N, D), jnp.float32),
    )(h1, h2, h3, iw1, ib1, iw2t)


def kernel(x, edge_index, confidences, params):
    src = edge_index[0]
    dst = edge_index[1]
    w = _edge_weights(confidences, src, dst)
    h = x
    outs = []
    for lp in params["layers"]:
        msg, slp = _dense_layer(h, lp["mw1"], lp["mb1"], lp["mw2"], lp["mb2"],
                                lp["sw1"], lp["sb1"], lp["sw2"], lp["sb2"])
        parts = _aggregate(msg, src, dst, w)
        h = _combine(parts[0], parts[1], slp)
        outs.append(h)
    return _importance(outs[0], outs[1], outs[2],
                       params["imp_w1"], params["imp_b1"].reshape(1, D),
                       params["imp_w2"].reshape(1, D))


# speedup vs baseline: 20.5474x; 1.0952x over previous
import functools

import jax
import jax.numpy as jnp
from jax import lax
from jax.experimental import pallas as pl
from jax.experimental.pallas import tpu as pltpu
from jax.experimental.pallas import tpu_sc as plsc

N = 10000
E = 320000
D = 128

NC = 2
NS = 16
NW = NC * NS
EPT = E // NW
RPT = 624
REM = N - NS * RPT
K = 80
NCHUNK = EPT // K

_mesh = plsc.VectorSubcoreMesh(core_axis_name="c", subcore_axis_name="s")
_sc_params = pltpu.CompilerParams(needs_layout_passes=False)


def _edge_weights(conf, src, dst):
    @functools.partial(
        pl.kernel,
        out_type=jax.ShapeDtypeStruct((E,), jnp.float32),
        mesh=_mesh,
        compiler_params=_sc_params,
        scratch_types=[
            pltpu.VMEM((N,), jnp.float32),
            pltpu.VMEM((EPT,), jnp.int32),
            pltpu.VMEM((EPT,), jnp.int32),
            pltpu.VMEM((EPT,), jnp.float32),
        ],
    )
    def kern(conf_hbm, src_hbm, dst_hbm, out_hbm, conf_v, src_v, dst_v, w_v):
        cid = lax.axis_index("c")
        sid = lax.axis_index("s")
        base = (sid * NC + cid) * EPT
        pltpu.sync_copy(conf_hbm, conf_v)
        pltpu.sync_copy(src_hbm.at[pl.ds(base, EPT)], src_v)
        pltpu.sync_copy(dst_hbm.at[pl.ds(base, EPT)], dst_v)

        def body(i, carry):
            sl = pl.ds(i * 16, 16)
            cs = plsc.load_gather(conf_v, [src_v[sl]])
            cd = plsc.load_gather(conf_v, [dst_v[sl]])
            w_v[sl] = jnp.exp(-jnp.abs(cs - cd))
            return carry

        lax.fori_loop(0, EPT // 16, body, 0)
        pltpu.sync_copy(w_v, out_hbm.at[pl.ds(base, EPT)])

    return kern(conf, src, dst)


def _aggregate(msg, src, dst3, w):
    @functools.partial(
        pl.kernel,
        out_type=jax.ShapeDtypeStruct((NC, N, D), jnp.float32),
        mesh=_mesh,
        compiler_params=_sc_params,
        scratch_types=[
            pltpu.VMEM_SHARED((N, D), jnp.float32),
            pltpu.VMEM((EPT,), jnp.int32),
            pltpu.VMEM((K, D), jnp.float32),
            pltpu.VMEM((K, D), jnp.float32),
            pltpu.VMEM((K, D), jnp.float32),
            pltpu.VMEM((K,), jnp.float32),
            pltpu.VMEM((K,), jnp.float32),
            pltpu.VMEM((K,), jnp.float32),
            pltpu.VMEM((K,), jnp.int32),
            pltpu.VMEM((K,), jnp.int32),
            pltpu.VMEM((K,), jnp.int32),
            pltpu.SemaphoreType.DMA,
            pltpu.SemaphoreType.DMA,
            pltpu.SemaphoreType.DMA,
            pltpu.SemaphoreType.DMA,
            pltpu.SemaphoreType.DMA,
            pltpu.SemaphoreType.DMA,
            pltpu.SemaphoreType.DMA,
            pltpu.SemaphoreType.DMA,
            pltpu.SemaphoreType.DMA,
        ],
    )
    def kern(msg_hbm, src_hbm, dst_hbm, w_hbm, out_hbm,
             acc, srcv, rows0, rows1, rows2, wb0, wb1, wb2, db0, db1, db2,
             gs0, gs1, gs2, ws0, ws1, ws2, ss0, ss1, ss2):
        cid = lax.axis_index("c")
        sid = lax.axis_index("s")
        wid = sid * NC + cid
        ebase = wid * EPT
        rows = (rows0, rows1, rows2)
        wb = (wb0, wb1, wb2)
        db = (db0, db1, db2)
        gs = (gs0, gs1, gs2)
        ws = (ws0, ws1, ws2)
        ss = (ss0, ss1, ss2)
        pltpu.sync_copy(src_hbm.at[pl.ds(ebase, EPT)], srcv)

        def zrow(i, carry):
            for c in range(D // 16):
                rows0[i, pl.ds(c * 16, 16)] = jnp.zeros((16,), jnp.float32)
            return carry

        lax.fori_loop(0, K, zrow, 0)
        for r in range(RPT // K):
            pltpu.sync_copy(rows0, acc.at[pl.ds(sid * RPT + r * K, K)])
        pltpu.sync_copy(rows0.at[pl.ds(0, RPT - (RPT // K) * K)],
                        acc.at[pl.ds(sid * RPT + (RPT // K) * K,
                                     RPT - (RPT // K) * K)])

        @pl.when(sid == NS - 1)
        def _():
            pltpu.sync_copy(rows0.at[pl.ds(0, REM)],
                            acc.at[pl.ds(NS * RPT, REM)])

        plsc.subcore_barrier()

        def start_fetch(j, s):
            pltpu.async_copy(msg_hbm.at[srcv.at[pl.ds(j * K, K)]],
                             rows[s], gs[s])
            pltpu.async_copy(w_hbm.at[pl.ds(ebase + j * K, K)], wb[s], ws[s])
            pltpu.async_copy(dst_hbm.at[pl.ds(ebase + j * K, K)], db[s], ws[s])

        def wait_fetch(j, s):
            pltpu.make_async_copy(msg_hbm.at[srcv.at[pl.ds(j * K, K)]],
                                  rows[s], gs[s]).wait()
            pltpu.make_async_copy(w_hbm.at[pl.ds(ebase + j * K, K)],
                                  wb[s], ws[s]).wait()
            pltpu.make_async_copy(dst_hbm.at[pl.ds(ebase + j * K, K)],
                                  db[s], ws[s]).wait()

        def start_scatter(s):
            pltpu.async_copy(rows[s], acc.at[db[s]], ss[s], add=True)

        def wait_scatter(s):
            pltpu.make_async_copy(rows[s], acc.at[db[s]], ss[s]).wait()

        def scale(s):
            def body(g, c2):
                wvec = wb[s][pl.ds(g * 16, 16)]
                for r in range(16):
                    wi = wvec[r]
                    for c in range(D // 16):
                        sl = pl.ds(c * 16, 16)
                        rows[s][g * 16 + r, sl] = rows[s][g * 16 + r, sl] * wi
                return c2

            lax.fori_loop(0, K // 16, body, 0)

        def step(j, s, first):
            wait_fetch(j, s)
            scale(s)
            start_scatter(s)
            if not first:
                wait_scatter((s + 2) % 3)

            @pl.when(j + 2 < NCHUNK)
            def _():
                start_fetch(j + 2, (s + 2) % 3)

        start_fetch(0, 0)
        start_fetch(1, 1)
        step(0, 0, True)
        step(1, 1, False)

        def triple(t, carry):
            j0 = 3 * t + 2
            step(j0, 2, False)
            step(j0 + 1, 0, False)
            step(j0 + 2, 1, False)
            return carry

        lax.fori_loop(0, (NCHUNK - 2) // 3, triple, 0)
        wait_scatter((NCHUNK - 1) % 3)

        plsc.subcore_barrier()
        rbase = sid * RPT
        pltpu.sync_copy(acc.at[pl.ds(rbase, RPT)],
                        out_hbm.at[cid, pl.ds(rbase, RPT)])

        @pl.when(sid == NS - 1)
        def _():
            pltpu.sync_copy(acc.at[pl.ds(NS * RPT, REM)],
                            out_hbm.at[cid, pl.ds(NS * RPT, REM)])

    return kern(msg, src, dst3, w)


_TC_R = 1000


def _dense_layer(h, mw1, mb1, mw2, mb2, sw1, sb1, sw2, sb2):
    def body(h_ref, mw1r, mb1r, mw2r, mb2r, sw1r, sb1r, sw2r, sb2r,
             msg_ref, sl_ref):
        hb = h_ref[...]
        hid = jnp.maximum(
            jnp.dot(hb, mw1r[...], preferred_element_type=jnp.float32)
            + mb1r[...], 0.0)
        msg_ref[...] = (
            jnp.dot(hid, mw2r[...], preferred_element_type=jnp.float32)
            + mb2r[...])
        shid = jnp.maximum(
            jnp.dot(hb, sw1r[...], preferred_element_type=jnp.float32)
            + sb1r[...], 0.0)
        sl_ref[...] = (
            jnp.dot(shid, sw2r[...], preferred_element_type=jnp.float32)
            + sb2r[...])

    row_spec = pl.BlockSpec((_TC_R, D), lambda i: (i, 0))
    w_spec = pl.BlockSpec((D, D), lambda i: (0, 0))
    b_spec = pl.BlockSpec((1, D), lambda i: (0, 0))
    return pl.pallas_call(
        body,
        grid=(N // _TC_R,),
        in_specs=[row_spec, w_spec, b_spec, w_spec, b_spec,
                  w_spec, b_spec, w_spec, b_spec],
        out_specs=[row_spec, row_spec],
        out_shape=[jax.ShapeDtypeStruct((N, D), jnp.float32),
                   jax.ShapeDtypeStruct((N, D), jnp.float32)],
    )(h, mw1, mb1.reshape(1, D), mw2, mb2.reshape(1, D),
      sw1, sb1.reshape(1, D), sw2, sb2.reshape(1, D))


def _layer_tc(parts, slp, mw1, mb1, mw2, mb2, sw1, sb1, sw2, sb2):
    def body(p_ref, slp_ref, mw1r, mb1r, mw2r, mb2r, sw1r, sb1r, sw2r, sb2r,
             h_ref, msg_ref, sl_ref):
        hb = jnp.maximum(p_ref[0] + p_ref[1] + slp_ref[...], 0.0)
        h_ref[...] = hb
        hid = jnp.maximum(
            jnp.dot(hb, mw1r[...], preferred_element_type=jnp.float32)
            + mb1r[...], 0.0)
        msg_ref[...] = (
            jnp.dot(hid, mw2r[...], preferred_element_type=jnp.float32)
            + mb2r[...])
        shid = jnp.maximum(
            jnp.dot(hb, sw1r[...], preferred_element_type=jnp.float32)
            + sb1r[...], 0.0)
        sl_ref[...] = (
            jnp.dot(shid, sw2r[...], preferred_element_type=jnp.float32)
            + sb2r[...])

    row_spec = pl.BlockSpec((_TC_R, D), lambda i: (i, 0))
    p_spec = pl.BlockSpec((2, _TC_R, D), lambda i: (0, i, 0))
    w_spec = pl.BlockSpec((D, D), lambda i: (0, 0))
    b_spec = pl.BlockSpec((1, D), lambda i: (0, 0))
    return pl.pallas_call(
        body,
        grid=(N // _TC_R,),
        in_specs=[p_spec, row_spec, w_spec, b_spec, w_spec, b_spec,
                  w_spec, b_spec, w_spec, b_spec],
        out_specs=[row_spec, row_spec, row_spec],
        out_shape=[jax.ShapeDtypeStruct((N, D), jnp.float32),
                   jax.ShapeDtypeStruct((N, D), jnp.float32),
                   jax.ShapeDtypeStruct((N, D), jnp.float32)],
    )(parts, slp, mw1, mb1.reshape(1, D), mw2, mb2.reshape(1, D),
      sw1, sb1.reshape(1, D), sw2, sb2.reshape(1, D))


def _importance(h1, h2, parts3, slp3, iw1, ib1, iw2t):
    def body(h1r, h2r, p3r, slp3r, w1r, b1r, w2r, outr):
        w1, b1, w2 = w1r[...], b1r[...], w2r[...]

        def score(hb):
            sh = jnp.maximum(
                jnp.dot(hb, w1, preferred_element_type=jnp.float32) + b1, 0.0)
            return jnp.sum(sh * w2, axis=1, keepdims=True)

        a, b = h1r[...], h2r[...]
        c = jnp.maximum(p3r[0] + p3r[1] + slp3r[...], 0.0)
        s1, s2, s3 = score(a), score(b), score(c)
        m = jnp.maximum(jnp.maximum(s1, s2), s3)
        e1 = jnp.exp(s1 - m)
        e2 = jnp.exp(s2 - m)
        e3 = jnp.exp(s3 - m)
        outr[...] = (e1 * a + e2 * b + e3 * c) / (e1 + e2 + e3)

    row_spec = pl.BlockSpec((_TC_R, D), lambda i: (i, 0))
    p_spec = pl.BlockSpec((2, _TC_R, D), lambda i: (0, i, 0))
    w_spec = pl.BlockSpec((D, D), lambda i: (0, 0))
    b_spec = pl.BlockSpec((1, D), lambda i: (0, 0))
    return pl.pallas_call(
        body,
        grid=(N // _TC_R,),
        in_specs=[row_spec, row_spec, p_spec, row_spec,
                  w_spec, b_spec, b_spec],
        out_specs=row_spec,
        out_shape=jax.ShapeDtypeStruct((N, D), jnp.float32),
    )(h1, h2, parts3, slp3, iw1, ib1, iw2t)


def kernel(x, edge_index, confidences, params):
    src = edge_index[0]
    dst = edge_index[1]
    w = _edge_weights(confidences, src, dst)
    lps = params["layers"]

    def wargs(lp):
        return (lp["mw1"], lp["mb1"], lp["mw2"], lp["mb2"],
                lp["sw1"], lp["sb1"], lp["sw2"], lp["sb2"])

    msg1, slp1 = _dense_layer(x, *wargs(lps[0]))
    parts1 = _aggregate(msg1, src, dst, w)
    h1, msg2, slp2 = _layer_tc(parts1, slp1, *wargs(lps[1]))
    parts2 = _aggregate(msg2, src, dst, w)
    h2, msg3, slp3 = _layer_tc(parts2, slp2, *wargs(lps[2]))
    parts3 = _aggregate(msg3, src, dst, w)
    return _importance(h1, h2, parts3, slp3,
                       params["imp_w1"], params["imp_b1"].reshape(1, D),
                       params["imp_w2"].reshape(1, D))
